# baseline (speedup 1.0000x reference)
import functools

import jax
import jax.numpy as jnp
from jax import lax
from jax.experimental import pallas as pl
from jax.experimental.pallas import tpu as pltpu
from jax.experimental.pallas import tpu_sc as plsc

B = 128
D = 768
F = 65536
K = 64

FC = 4096
RB = 8


def _enc_body(x_ref, pb_ref, w_ref, b_ref, z_ref):
    xc = x_ref[...] - pb_ref[...][None, :]
    z = lax.dot_general(xc, w_ref[...], (((1,), (1,)), ((), ())),
                        precision=lax.Precision.DEFAULT,
                        preferred_element_type=jnp.float32)
    z_ref[...] = z + b_ref[...][0][None, :]


def _encode(x, pre_bias, W_enc, b_enc):
    grid = F // FC
    return pl.pallas_call(
        _enc_body,
        grid=(grid,),
        in_specs=[
            pl.BlockSpec((B, D), lambda i: (0, 0)),
            pl.BlockSpec((D,), lambda i: (0,)),
            pl.BlockSpec((FC, D), lambda i: (i, 0)),
            pl.BlockSpec((1, FC), lambda i: (0, i)),
        ],
        out_specs=pl.BlockSpec((B, FC), lambda i: (0, i)),
        out_shape=jax.ShapeDtypeStruct((B, F), jnp.float32),
    )(x, pre_bias, W_enc, b_enc.reshape(1, F))


def _topk_body(z_ref, sparse_ref):
    z = z_ref[...]
    b = lax.bitcast_convert_type(z, jnp.int32)
    m = (b >> 31) | jnp.int32(-2147483648)
    u = lax.bitcast_convert_type(b ^ m, jnp.uint32)

    def bit_step(i, t):
        cand = t | (jnp.uint32(1) << (jnp.uint32(31) - i.astype(jnp.uint32)))
        cnt = jnp.sum((u >= cand[:, None]).astype(jnp.float32), axis=1)
        return jnp.where(cnt >= K, cand, t)

    T = lax.fori_loop(0, 32, bit_step, jnp.zeros((RB,), jnp.uint32))
    Tb = T[:, None]

    gt = u > Tb
    eq = u == Tb
    n_gt = jnp.sum(gt.astype(jnp.float32), axis=1, keepdims=True)
    need_eq = jnp.float32(K) - n_gt

    eqf = eq.astype(jnp.float32).reshape(RB * 512, 128)
    r_i = lax.broadcasted_iota(jnp.int32, (128, 128), 0)
    c_i = lax.broadcasted_iota(jnp.int32, (128, 128), 1)
    incl = (r_i <= c_i).astype(jnp.float32)
    within = lax.dot_general(eqf, incl, (((1,), (0,)), ((), ())),
                             precision=lax.Precision.HIGHEST,
                             preferred_element_type=jnp.float32)
    chunk_tot = jnp.sum(eqf, axis=1).reshape(RB, 512)
    r2 = lax.broadcasted_iota(jnp.int32, (512, 512), 0)
    c2 = lax.broadcasted_iota(jnp.int32, (512, 512), 1)
    strict = (r2 < c2).astype(jnp.float32)
    offs = lax.dot_general(chunk_tot, strict, (((1,), (0,)), ((), ())),
                           precision=lax.Precision.HIGHEST,
                           preferred_element_type=jnp.float32)
    rank = (within.reshape(RB, 512, 128)
            + offs[:, :, None]).reshape(RB, F)

    sel = gt | (eq & (rank <= need_eq))
    sparse_ref[...] = jnp.where(sel, jnp.maximum(z, 0.0), 0.0)


def _topk_sparse(z):
    grid = B // RB
    return pl.pallas_call(
        _topk_body,
        grid=(grid,),
        in_specs=[pl.BlockSpec((RB, F), lambda i: (i, 0))],
        out_specs=pl.BlockSpec((RB, F), lambda i: (i, 0)),
        out_shape=jax.ShapeDtypeStruct((B, F), jnp.float32),
    )(z)


CAP = 2048
HALF = 16384

_MINI = -2147483648


def _monot(v):
    bi = lax.bitcast_convert_type(v, jnp.int32)
    m = (bi >> 31) | _MINI
    return lax.bitcast_convert_type(bi ^ m, jnp.uint32)


def _inv_monot(k):
    ki = lax.bitcast_convert_type(k, jnp.int32)
    bi = jnp.where(ki < 0, ki ^ _MINI, ~ki)
    return lax.bitcast_convert_type(bi, jnp.float32)


def _sc_topk(z):
    NC, NS = 2, 16
    NW = NC * NS
    rows_per_w = B // NW
    mesh = plsc.VectorSubcoreMesh(core_axis_name="c", subcore_axis_name="s",
                                  num_cores=NC, num_subcores=NS)

    @functools.partial(
        pl.kernel,
        out_type=jax.ShapeDtypeStruct((B, F), jnp.float32),
        mesh=mesh,
        compiler_params=pltpu.CompilerParams(needs_layout_passes=False),
        scratch_types=[
            pltpu.VMEM((F,), jnp.float32),
            pltpu.VMEM((4224,), jnp.int32),
            pltpu.VMEM((256,), jnp.int32),
            pltpu.VMEM((CAP,), jnp.int32),
            pltpu.VMEM((CAP,), jnp.int32),
            pltpu.VMEM((K,), jnp.int32),
            pltpu.VMEM((K,), jnp.int32),
            pltpu.VMEM((HALF,), jnp.float32),
            pltpu.VMEM((K,), jnp.float32),
            pltpu.VMEM((K,), jnp.int32),
            pltpu.SemaphoreType.DMA,
            pltpu.SemaphoreType.DMA,
        ],
    )
    def body(z_hbm, out_hbm, rowb, hist, sbuf, ck, ci, selk, seli, zbuf,
             selv, idxg, zsem, ssem):
        wid = lax.axis_index("s") * NC + lax.axis_index("c")
        lane = lax.iota(jnp.int32, 16)
        lane257 = lane * 257
        ones16 = jnp.ones((16,), jnp.int32)
        zero16f = jnp.zeros((16,), jnp.float32)

        def zero_hist():
            @plsc.parallel_loop(0, 264, unroll=8)
            def _zh(i):
                hist[pl.ds(i * 16, 16)] = jnp.zeros((16,), jnp.int32)

        @plsc.parallel_loop(0, HALF // 16, unroll=8)
        def _zb0(i):
            zbuf[pl.ds(i * 16, 16)] = zero16f

        def scan_hist(need):
            def sb(i, c):
                carry, bcount = c
                g = 15 - i
                acc = hist[pl.ds(g * 16, 16)]
                for l in range(1, 16):
                    acc = acc + hist[pl.ds(l * 257 + g * 16, 16)]
                cs = plsc.cumsum(lax.rev(acc, (0,)))
                sg = lax.rev(cs + carry, (0,))
                sbuf[pl.ds(g * 16, 16)] = sg
                carry = carry + jnp.max(cs)
                bcount = bcount + plsc.all_reduce_population_count(sg >= need)
                return carry, bcount
            _, bcount = lax.fori_loop(
                0, 16, sb, (jnp.int32(0), jnp.zeros((16,), jnp.int32)))
            bstar = jnp.max(bcount) - 1
            up = jnp.minimum(bstar + 1, 255)
            sa_v = plsc.load_gather(sbuf, [jnp.broadcast_to(up, (16,))])
            ca_v = plsc.load_gather(sbuf, [jnp.broadcast_to(bstar, (16,))])
            s_above = jnp.where(bstar >= 255, 0, jnp.max(sa_v))
            return bstar, s_above, jnp.max(ca_v)

        def hist_cand(nv, cnt, lo, hi, shift):
            zero_hist()
            sh = jnp.uint32(shift)

            @plsc.parallel_loop(0, nv, unroll=4)
            def _hc(i):
                base = i * 16
                k = lax.bitcast_convert_type(ck[pl.ds(base, 16)],
                                             jnp.uint32)
                valid = (base + lane) < cnt
                m = valid & (k >= lo) & (k < hi)
                b = lax.bitcast_convert_type((k - lo) >> sh,
                                             jnp.int32) & 255
                plsc.addupdate_scatter(hist, [lane257 + b], ones16, mask=m)

        def hist_row(lo, hi, shift):
            zero_hist()
            sh = jnp.uint32(shift)

            @plsc.parallel_loop(0, 4096, unroll=8)
            def _hr(i):
                k = _monot(rowb[pl.ds(i * 16, 16)])
                m = (k >= lo) & (k < hi)
                b = lax.bitcast_convert_type((k - lo) >> sh,
                                             jnp.int32) & 255
                plsc.addupdate_scatter(hist, [lane257 + b], ones16, mask=m)

        def refine(level1, use_cand, nv, cnt):
            b1, sa1, _ = level1
            lo = b1.astype(jnp.uint32) << 24
            need = jnp.int32(K) - sa1
            width = jnp.uint32(1) << 24
            for shift in (16, 8, 0):
                if use_cand:
                    hist_cand(nv, cnt, lo, lo + width, shift)
                else:
                    hist_row(lo, lo + width, shift)
                bl, sal, _ = scan_hist(need)
                lo = lo + (bl.astype(jnp.uint32) << shift)
                need = need - sal
                width = jnp.uint32(1) << shift
            return lo, need

        def final_select(use_cand, nv, cnt, t_u, need_eq):
            def step(base, eqc, selc):
                if use_cand:
                    k = lax.bitcast_convert_type(ck[pl.ds(base, 16)], jnp.uint32)
                    idx = ci[pl.ds(base, 16)]
                    valid = (base + lane) < cnt
                else:
                    k = _monot(rowb[pl.ds(base, 16)])
                    idx = base + lane
                    valid = (base + lane) < F
                gt = valid & (k > t_u)
                eq = valid & (k == t_u)
                eqr = eqc + plsc.cumsum(eq.astype(jnp.int32))
                sel = gt | (eq & (eqr <= need_eq))
                addr = selc + plsc.cumsum(sel.astype(jnp.int32)) - 1
                plsc.store_scatter(selk, [addr],
                                   lax.bitcast_convert_type(k, jnp.int32), mask=sel)
                plsc.store_scatter(seli, [addr], idx, mask=sel)
                eqc = eqc + plsc.all_reduce_population_count(eq)
                selc = selc + plsc.all_reduce_population_count(sel)
                return eqc, selc
            z16 = jnp.zeros((16,), jnp.int32)

            @plsc.parallel_loop(0, nv, unroll=4, carry=(z16, z16))
            def _fs(i, c):
                eqc, selc = c
                return step(i * 16, eqc, selc)

        def row_body(r, lspec):
            row = wid * rows_per_w + r
            qdescs = [
                pltpu.async_copy(z_hbm.at[row, pl.ds(q * 16384, 16384)],
                                 rowb.at[pl.ds(q * 16384, 16384)], zsem)
                for q in range(4)
            ]

            zero_hist()
            carry = jnp.zeros((16,), jnp.int32)
            for q in range(4):
                qdescs[q].wait()

                @plsc.parallel_loop(0, 1024, unroll=8, carry=carry)
                def _h1(i, c, _q=q):
                    base = _q * 16384 + i * 16
                    k = _monot(rowb[pl.ds(base, 16)])
                    b = lax.bitcast_convert_type(k >> jnp.uint32(24),
                                                 jnp.int32)
                    plsc.addupdate_scatter(hist, [lane257 + b], ones16,
                                           mask=ones16 > 0)
                    m = k >= lspec
                    addr = jnp.minimum(c + plsc.cumsum(m.astype(jnp.int32)),
                                       CAP) - 1
                    plsc.store_scatter(
                        ck, [addr],
                        lax.bitcast_convert_type(k, jnp.int32), mask=m)
                    plsc.store_scatter(ci, [addr], base + lane, mask=m)
                    return c + plsc.all_reduce_population_count(m)
                carry = _h1
            spec_cnt = jnp.max(carry)

            lvl1 = scan_hist(jnp.int32(K))
            b1, _, c1 = lvl1
            lo1 = b1.astype(jnp.uint32) << 24
            spec_ok = (lspec <= lo1) & (spec_cnt <= CAP)

            @pl.when(spec_ok)
            def _spec():
                nv = (spec_cnt + 15) // 16
                t_u, need_eq = refine(lvl1, True, nv, spec_cnt)
                final_select(True, nv, spec_cnt, t_u, need_eq)

            @pl.when(jnp.logical_not(spec_ok) & (c1 <= CAP))
            def _fast():
                @plsc.parallel_loop(0, 4096, unroll=8,
                                    carry=jnp.zeros((16,), jnp.int32))
                def _cb(i, carry):
                    base = i * 16
                    k = _monot(rowb[pl.ds(base, 16)])
                    m = k >= lo1
                    addr = carry + plsc.cumsum(m.astype(jnp.int32)) - 1
                    plsc.store_scatter(
                        ck, [addr],
                        lax.bitcast_convert_type(k, jnp.int32), mask=m)
                    plsc.store_scatter(ci, [addr], base + lane, mask=m)
                    return carry + plsc.all_reduce_population_count(m)
                nv = (c1 + 15) // 16
                t_u, need_eq = refine(lvl1, True, nv, c1)
                final_select(True, nv, c1, t_u, need_eq)

            @pl.when(jnp.logical_not(spec_ok) & (c1 > CAP))
            def _slow():
                t_u, need_eq = refine(lvl1, False, 4096, jnp.int32(F))
                final_select(False, 4096, jnp.int32(F), t_u, need_eq)

            for q in range(F // HALF):
                sc = []
                for j in range(K // 16):
                    kv = lax.bitcast_convert_type(selk[pl.ds(j * 16, 16)],
                                                  jnp.uint32)
                    val = jnp.maximum(_inv_monot(kv), 0.0)
                    idx = seli[pl.ds(j * 16, 16)]
                    inb = (idx >= q * HALF) & (idx < (q + 1) * HALF)
                    locc = (idx - q * HALF) & (HALF - 1)
                    plsc.store_scatter(zbuf, [locc], val, mask=inb)
                    sc.append((locc, inb))
                pltpu.sync_copy(zbuf, out_hbm.at[row, pl.ds(q * HALF, HALF)])
                for locc, inb in sc:
                    plsc.store_scatter(zbuf, [locc], zero16f, mask=inb)
            return jnp.where(b1 >= 1, lo1 - (jnp.uint32(1) << 24),
                             jnp.uint32(0))

        lax.fori_loop(0, rows_per_w, row_body,
                      jnp.uint32(0xFFFFFFFF))

    return body(z)


def _dec_body(s_ref, w_ref, pb_ref, out_ref):
    i = pl.program_id(0)
    part = lax.dot_general(s_ref[...], w_ref[...], (((1,), (1,)), ((), ())),
                           precision=lax.Precision.DEFAULT,
                           preferred_element_type=jnp.float32)

    @pl.when(i == 0)
    def _init():
        out_ref[...] = part + pb_ref[...][None, :]

    @pl.when(i != 0)
    def _acc():
        out_ref[...] += part


def _decode(sparse, W_dec, pre_bias):
    grid = F // FC
    return pl.pallas_call(
        _dec_body,
        grid=(grid,),
        in_specs=[
            pl.BlockSpec((B, FC), lambda i: (0, i)),
            pl.BlockSpec((D, FC), lambda i: (0, i)),
            pl.BlockSpec((D,), lambda i: (0,)),
        ],
        out_specs=pl.BlockSpec((B, D), lambda i: (0, 0)),
        out_shape=jax.ShapeDtypeStruct((B, D), jnp.float32),
    )(sparse, W_dec, pre_bias)


def kernel(x, pre_bias, W_enc, b_enc, W_dec):
    z = _encode(x, pre_bias, W_enc, b_enc)
    sparse = _sc_topk(z)
    x_hat = _decode(sparse, W_dec, pre_bias)
    return (x_hat, sparse)


# speedup vs baseline: 1.1022x; 1.1022x over previous
import functools

import jax
import jax.numpy as jnp
from jax import lax
from jax.experimental import pallas as pl
from jax.experimental.pallas import tpu as pltpu
from jax.experimental.pallas import tpu_sc as plsc

B = 128
D = 768
F = 65536
K = 64

FC = 4096
RB = 8


def _enc_body(x_ref, pb_ref, w_ref, b_ref, z_ref):
    xc = x_ref[...] - pb_ref[...][None, :]
    z = lax.dot_general(xc, w_ref[...], (((1,), (1,)), ((), ())),
                        precision=lax.Precision.DEFAULT,
                        preferred_element_type=jnp.float32)
    z_ref[...] = z + b_ref[...][0][None, :]


def _encode(x, pre_bias, W_enc, b_enc):
    grid = F // FC
    return pl.pallas_call(
        _enc_body,
        grid=(grid,),
        in_specs=[
            pl.BlockSpec((B, D), lambda i: (0, 0)),
            pl.BlockSpec((D,), lambda i: (0,)),
            pl.BlockSpec((FC, D), lambda i: (i, 0)),
            pl.BlockSpec((1, FC), lambda i: (0, i)),
        ],
        out_specs=pl.BlockSpec((B, FC), lambda i: (0, i)),
        out_shape=jax.ShapeDtypeStruct((B, F), jnp.float32),
    )(x, pre_bias, W_enc, b_enc.reshape(1, F))


def _topk_body(z_ref, sparse_ref):
    z = z_ref[...]
    b = lax.bitcast_convert_type(z, jnp.int32)
    m = (b >> 31) | jnp.int32(-2147483648)
    u = lax.bitcast_convert_type(b ^ m, jnp.uint32)

    def bit_step(i, t):
        cand = t | (jnp.uint32(1) << (jnp.uint32(31) - i.astype(jnp.uint32)))
        cnt = jnp.sum((u >= cand[:, None]).astype(jnp.float32), axis=1)
        return jnp.where(cnt >= K, cand, t)

    T = lax.fori_loop(0, 32, bit_step, jnp.zeros((RB,), jnp.uint32))
    Tb = T[:, None]

    gt = u > Tb
    eq = u == Tb
    n_gt = jnp.sum(gt.astype(jnp.float32), axis=1, keepdims=True)
    need_eq = jnp.float32(K) - n_gt

    eqf = eq.astype(jnp.float32).reshape(RB * 512, 128)
    r_i = lax.broadcasted_iota(jnp.int32, (128, 128), 0)
    c_i = lax.broadcasted_iota(jnp.int32, (128, 128), 1)
    incl = (r_i <= c_i).astype(jnp.float32)
    within = lax.dot_general(eqf, incl, (((1,), (0,)), ((), ())),
                             precision=lax.Precision.HIGHEST,
                             preferred_element_type=jnp.float32)
    chunk_tot = jnp.sum(eqf, axis=1).reshape(RB, 512)
    r2 = lax.broadcasted_iota(jnp.int32, (512, 512), 0)
    c2 = lax.broadcasted_iota(jnp.int32, (512, 512), 1)
    strict = (r2 < c2).astype(jnp.float32)
    offs = lax.dot_general(chunk_tot, strict, (((1,), (0,)), ((), ())),
                           precision=lax.Precision.HIGHEST,
                           preferred_element_type=jnp.float32)
    rank = (within.reshape(RB, 512, 128)
            + offs[:, :, None]).reshape(RB, F)

    sel = gt | (eq & (rank <= need_eq))
    sparse_ref[...] = jnp.where(sel, jnp.maximum(z, 0.0), 0.0)


def _topk_sparse(z):
    grid = B // RB
    return pl.pallas_call(
        _topk_body,
        grid=(grid,),
        in_specs=[pl.BlockSpec((RB, F), lambda i: (i, 0))],
        out_specs=pl.BlockSpec((RB, F), lambda i: (i, 0)),
        out_shape=jax.ShapeDtypeStruct((B, F), jnp.float32),
    )(z)


CAP = 2048
HALF = 16384

_MINI = -2147483648


def _monot(v):
    bi = lax.bitcast_convert_type(v, jnp.int32)
    m = (bi >> 31) | _MINI
    return lax.bitcast_convert_type(bi ^ m, jnp.uint32)


def _inv_monot(k):
    ki = lax.bitcast_convert_type(k, jnp.int32)
    bi = jnp.where(ki < 0, ki ^ _MINI, ~ki)
    return lax.bitcast_convert_type(bi, jnp.float32)


def _sc_topk(z):
    NC, NS = 2, 16
    NW = NC * NS
    rows_per_w = B // NW
    mesh = plsc.VectorSubcoreMesh(core_axis_name="c", subcore_axis_name="s",
                                  num_cores=NC, num_subcores=NS)

    @functools.partial(
        pl.kernel,
        out_type=jax.ShapeDtypeStruct((B, F), jnp.float32),
        mesh=mesh,
        compiler_params=pltpu.CompilerParams(needs_layout_passes=False),
        scratch_types=[
            pltpu.VMEM((F,), jnp.float32),
            pltpu.VMEM((4224,), jnp.int32),
            pltpu.VMEM((256,), jnp.int32),
            pltpu.VMEM((CAP,), jnp.int32),
            pltpu.VMEM((CAP,), jnp.int32),
            pltpu.VMEM((K,), jnp.int32),
            pltpu.VMEM((K,), jnp.int32),
            pltpu.VMEM((HALF,), jnp.float32),
            pltpu.VMEM((K,), jnp.float32),
            pltpu.VMEM((K,), jnp.int32),
            pltpu.SemaphoreType.DMA,
            pltpu.SemaphoreType.DMA,
        ],
    )
    def body(z_hbm, out_hbm, rowb, hist, sbuf, ck, ci, selk, seli, zbuf,
             selv, idxg, zsem, ssem):
        wid = lax.axis_index("s") * NC + lax.axis_index("c")
        lane = lax.iota(jnp.int32, 16)
        lane257 = lane * 257
        ones16 = jnp.ones((16,), jnp.int32)
        zero16f = jnp.zeros((16,), jnp.float32)

        def zero_hist():
            @plsc.parallel_loop(0, 264, unroll=8)
            def _zh(i):
                hist[pl.ds(i * 16, 16)] = jnp.zeros((16,), jnp.int32)

        @plsc.parallel_loop(0, HALF // 16, unroll=8)
        def _zb0(i):
            zbuf[pl.ds(i * 16, 16)] = zero16f

        def scan_hist(need):
            def sb(i, c):
                carry, bcount = c
                g = 15 - i
                acc = hist[pl.ds(g * 16, 16)]
                for l in range(1, 16):
                    acc = acc + hist[pl.ds(l * 257 + g * 16, 16)]
                cs = plsc.cumsum(lax.rev(acc, (0,)))
                sg = lax.rev(cs + carry, (0,))
                sbuf[pl.ds(g * 16, 16)] = sg
                carry = carry + jnp.max(cs)
                bcount = bcount + plsc.all_reduce_population_count(sg >= need)
                return carry, bcount
            _, bcount = lax.fori_loop(
                0, 16, sb, (jnp.int32(0), jnp.zeros((16,), jnp.int32)))
            bstar = jnp.max(bcount) - 1
            up = jnp.minimum(bstar + 1, 255)
            sa_v = plsc.load_gather(sbuf, [jnp.broadcast_to(up, (16,))])
            ca_v = plsc.load_gather(sbuf, [jnp.broadcast_to(bstar, (16,))])
            s_above = jnp.where(bstar >= 255, 0, jnp.max(sa_v))
            return bstar, s_above, jnp.max(ca_v)

        def hist_cand(nv, cnt, lo, hi, shift):
            zero_hist()
            sh = jnp.uint32(shift)

            @plsc.parallel_loop(0, nv, unroll=4)
            def _hc(i):
                base = i * 16
                idx = ci[pl.ds(base, 16)]
                k = _monot(plsc.load_gather(rowb, [idx]))
                valid = (base + lane) < cnt
                m = valid & (k >= lo) & (k < hi)
                b = lax.bitcast_convert_type((k - lo) >> sh,
                                             jnp.int32) & 255
                plsc.addupdate_scatter(hist, [lane257 + b], ones16, mask=m)

        def hist_row(lo, hi, shift):
            zero_hist()
            sh = jnp.uint32(shift)

            @plsc.parallel_loop(0, 4096, unroll=8)
            def _hr(i):
                k = _monot(rowb[pl.ds(i * 16, 16)])
                m = (k >= lo) & (k < hi)
                b = lax.bitcast_convert_type((k - lo) >> sh,
                                             jnp.int32) & 255
                plsc.addupdate_scatter(hist, [lane257 + b], ones16, mask=m)

        def refine(level1, use_cand, nv, cnt):
            b1, sa1, _ = level1
            lo = b1.astype(jnp.uint32) << 24
            need = jnp.int32(K) - sa1
            width = jnp.uint32(1) << 24
            for shift in (16, 8, 0):
                if use_cand:
                    hist_cand(nv, cnt, lo, lo + width, shift)
                else:
                    hist_row(lo, lo + width, shift)
                bl, sal, _ = scan_hist(need)
                lo = lo + (bl.astype(jnp.uint32) << shift)
                need = need - sal
                width = jnp.uint32(1) << shift
            return lo, need

        def final_select(use_cand, nv, cnt, t_u, need_eq):
            def step(base, eqc, selc):
                if use_cand:
                    idx = ci[pl.ds(base, 16)]
                    k = _monot(plsc.load_gather(rowb, [idx]))
                    valid = (base + lane) < cnt
                else:
                    k = _monot(rowb[pl.ds(base, 16)])
                    idx = base + lane
                    valid = (base + lane) < F
                gt = valid & (k > t_u)
                eq = valid & (k == t_u)
                eqr = eqc + plsc.cumsum(eq.astype(jnp.int32))
                sel = gt | (eq & (eqr <= need_eq))
                addr = selc + plsc.cumsum(sel.astype(jnp.int32)) - 1
                plsc.store_scatter(selk, [addr],
                                   lax.bitcast_convert_type(k, jnp.int32), mask=sel)
                plsc.store_scatter(seli, [addr], idx, mask=sel)
                eqc = eqc + plsc.all_reduce_population_count(eq)
                selc = selc + plsc.all_reduce_population_count(sel)
                return eqc, selc
            z16 = jnp.zeros((16,), jnp.int32)

            @plsc.parallel_loop(0, nv, unroll=4, carry=(z16, z16))
            def _fs(i, c):
                eqc, selc = c
                return step(i * 16, eqc, selc)

        def row_body(r, _):
            row = wid * rows_per_w + r
            qdescs = [
                pltpu.async_copy(z_hbm.at[row, pl.ds(q * 16384, 16384)],
                                 rowb.at[pl.ds(q * 16384, 16384)], zsem)
                for q in range(4)
            ]

            zero_hist()
            for q in range(4):
                qdescs[q].wait()

                @plsc.parallel_loop(0, 1024, unroll=8)
                def _h1(i, _q=q):
                    k = _monot(rowb[pl.ds(_q * 16384 + i * 16, 16)])
                    b = lax.bitcast_convert_type(k >> jnp.uint32(24),
                                                 jnp.int32)
                    plsc.addupdate_scatter(hist, [lane257 + b], ones16,
                                           mask=ones16 > 0)

            lvl1 = scan_hist(jnp.int32(K))
            b1, _, c1 = lvl1
            lo1 = b1.astype(jnp.uint32) << 24

            @pl.when(c1 <= CAP)
            def _fast():
                @plsc.parallel_loop(0, 4096, unroll=8,
                                    carry=jnp.zeros((16,), jnp.int32))
                def _cb(i, carry):
                    base = i * 16
                    k = _monot(rowb[pl.ds(base, 16)])
                    m = k >= lo1
                    addr = carry + plsc.cumsum(m.astype(jnp.int32)) - 1
                    plsc.store_scatter(ci, [addr], base + lane, mask=m)
                    return carry + plsc.all_reduce_population_count(m)
                nv = (c1 + 15) // 16
                t_u, need_eq = refine(lvl1, True, nv, c1)
                final_select(True, nv, c1, t_u, need_eq)

            @pl.when(c1 > CAP)
            def _slow():
                t_u, need_eq = refine(lvl1, False, 4096, jnp.int32(F))
                final_select(False, 4096, jnp.int32(F), t_u, need_eq)

            for q in range(F // HALF):
                sc = []
                for j in range(K // 16):
                    kv = lax.bitcast_convert_type(selk[pl.ds(j * 16, 16)],
                                                  jnp.uint32)
                    val = jnp.maximum(_inv_monot(kv), 0.0)
                    idx = seli[pl.ds(j * 16, 16)]
                    inb = (idx >= q * HALF) & (idx < (q + 1) * HALF)
                    locc = (idx - q * HALF) & (HALF - 1)
                    plsc.store_scatter(zbuf, [locc], val, mask=inb)
                    sc.append((locc, inb))
                pltpu.sync_copy(zbuf, out_hbm.at[row, pl.ds(q * HALF, HALF)])
                for locc, inb in sc:
                    plsc.store_scatter(zbuf, [locc], zero16f, mask=inb)
            return 0

        lax.fori_loop(0, rows_per_w, row_body, 0)

    return body(z)


def _dec_body(s_ref, w_ref, pb_ref, out_ref):
    i = pl.program_id(0)
    part = lax.dot_general(s_ref[...], w_ref[...], (((1,), (1,)), ((), ())),
                           precision=lax.Precision.DEFAULT,
                           preferred_element_type=jnp.float32)

    @pl.when(i == 0)
    def _init():
        out_ref[...] = part + pb_ref[...][None, :]

    @pl.when(i != 0)
    def _acc():
        out_ref[...] += part


def _decode(sparse, W_dec, pre_bias):
    grid = F // FC
    return pl.pallas_call(
        _dec_body,
        grid=(grid,),
        in_specs=[
            pl.BlockSpec((B, FC), lambda i: (0, i)),
            pl.BlockSpec((D, FC), lambda i: (0, i)),
            pl.BlockSpec((D,), lambda i: (0,)),
        ],
        out_specs=pl.BlockSpec((B, D), lambda i: (0, 0)),
        out_shape=jax.ShapeDtypeStruct((B, D), jnp.float32),
    )(sparse, W_dec, pre_bias)


def kernel(x, pre_bias, W_enc, b_enc, W_dec):
    z = _encode(x, pre_bias, W_enc, b_enc)
    sparse = _sc_topk(z)
    x_hat = _decode(sparse, W_dec, pre_bias)
    return (x_hat, sparse)
